# baseline (speedup 1.0000x reference)
import functools

import jax
import jax.numpy as jnp
from jax import lax
from jax.experimental import pallas as pl
from jax.experimental.pallas import tpu as pltpu
from jax.experimental.pallas import tpu_sc as plsc

N, D, H, C, E = 10000, 128, 128, 64, 320000

NC, NS = 2, 16
NW = NC * NS
NP = 10112
CH = 64
NBUF = 4
CPT = 160
EPT = CPT * CH
EP = NW * EPT
RPT = NP // NS
HID = 40

_mesh = plsc.VectorSubcoreMesh(
    core_axis_name="c", subcore_axis_name="s", num_cores=NC, num_subcores=NS
)


@functools.partial(
    pl.kernel,
    out_type=jax.ShapeDtypeStruct((NW, NP), jnp.float32),
    mesh=_mesh,
    scratch_types=[
        pltpu.VMEM((EPT,), jnp.int32),
        pltpu.VMEM((NP,), jnp.float32),
    ],
    compiler_params=pltpu.CompilerParams(needs_layout_passes=False),
)
def _deg_kernel(dst_hbm, out_hbm, dst_v, deg_v):
    c = lax.axis_index("c")
    s = lax.axis_index("s")
    wid = s * NC + c
    pltpu.sync_copy(dst_hbm.at[wid], dst_v)

    zeros16 = jnp.zeros((16,), jnp.float32)

    def zero_body(i, carry):
        deg_v[pl.ds(i * 16, 16)] = zeros16
        return carry

    lax.fori_loop(0, NP // 16, zero_body, 0)

    ones16 = jnp.ones((16,), jnp.float32)

    def scat_body(i, carry):
        idx = dst_v[pl.ds(i * 16, 16)]
        plsc.addupdate_scatter(deg_v, [idx], ones16)
        return carry

    lax.fori_loop(0, EPT // 16, scat_body, 0)
    pltpu.sync_copy(deg_v, out_hbm.at[wid])


@functools.partial(
    pl.kernel,
    out_type=jax.ShapeDtypeStruct((NC, NP, H), jnp.float32),
    mesh=_mesh,
    scratch_types=[
        pltpu.VMEM((HID, CH), jnp.int32),
        pltpu.VMEM((HID, CH), jnp.int32),
        pltpu.VMEM((NBUF, CH, H), jnp.float32),
        pltpu.VMEM_SHARED((NP, H), jnp.float32),
        [pltpu.SemaphoreType.DMA] * NBUF,
    ],
)
def _agg_kernel(y_hbm, src_hbm, dst_hbm, zeros_hbm, out_hbm,
                src_v, dst_v, rows_v, accum, sems):
    c = lax.axis_index("c")
    s = lax.axis_index("s")
    wid = s * NC + c
    pltpu.sync_copy(zeros_hbm.at[pl.ds(s * RPT, RPT)],
                    accum.at[pl.ds(s * RPT, RPT)])
    plsc.subcore_barrier()

    for half in range(CPT // HID):
        pltpu.sync_copy(src_hbm.at[wid, pl.ds(half * HID, HID)], src_v)
        pltpu.sync_copy(dst_hbm.at[wid, pl.ds(half * HID, HID)], dst_v)

        for b in range(NBUF):
            pltpu.async_copy(y_hbm.at[src_v.at[b]], rows_v.at[b], sems[b])

        def body(t, carry):
            for b in range(NBUF):
                j = NBUF * t + b
                pltpu.make_async_copy(y_hbm.at[src_v.at[j]], rows_v.at[b],
                                      sems[b]).wait()
                pltpu.sync_copy(rows_v.at[b], accum.at[dst_v.at[j]], add=True)

                @pl.when(j + NBUF < HID)
                def _():
                    pltpu.async_copy(y_hbm.at[src_v.at[j + NBUF]],
                                     rows_v.at[b], sems[b])

            return carry

        lax.fori_loop(0, HID // NBUF, body, 0)
    plsc.subcore_barrier()
    pltpu.sync_copy(accum.at[pl.ds(s * RPT, RPT)],
                    out_hbm.at[c, pl.ds(s * RPT, RPT)])


def _dinv_body(parts_ref, o_ref):
    deg = 1.0 + jnp.sum(parts_ref[...], axis=0)
    o_ref[...] = lax.rsqrt(deg)


_dinv_call = pl.pallas_call(
    _dinv_body,
    out_shape=jax.ShapeDtypeStruct((NP,), jnp.float32),
)

_GRID = NP // 128


def _dot(a, b):
    return jnp.dot(a, b, preferred_element_type=jnp.float32,
                   precision=lax.Precision.HIGHEST)


def _y1_body(x_ref, dinv_ref, w_ref, o_ref):
    o_ref[...] = _dot(x_ref[...] * dinv_ref[...], w_ref[...])


_y1_call = pl.pallas_call(
    _y1_body,
    grid=(_GRID,),
    in_specs=[
        pl.BlockSpec((128, D), lambda j: (j, 0)),
        pl.BlockSpec((128, 1), lambda j: (j, 0)),
        pl.BlockSpec((D, H), lambda j: (0, 0)),
    ],
    out_specs=pl.BlockSpec((128, H), lambda j: (j, 0)),
    out_shape=jax.ShapeDtypeStruct((NP, H), jnp.float32),
)


def _hy2_body(p_ref, y1_ref, dinv_ref, b1_ref, w2_ref, o_ref):
    dv = dinv_ref[...]
    h = jnp.maximum(dv * (p_ref[0] + p_ref[1] + y1_ref[...]) + b1_ref[...], 0.0)
    o_ref[...] = _dot(h * dv, w2_ref[...])


_hy2_call = pl.pallas_call(
    _hy2_body,
    grid=(_GRID,),
    in_specs=[
        pl.BlockSpec((NC, 128, H), lambda j: (0, j, 0)),
        pl.BlockSpec((128, H), lambda j: (j, 0)),
        pl.BlockSpec((128, 1), lambda j: (j, 0)),
        pl.BlockSpec((1, H), lambda j: (0, 0)),
        pl.BlockSpec((H, H), lambda j: (0, 0)),
    ],
    out_specs=pl.BlockSpec((128, H), lambda j: (j, 0)),
    out_shape=jax.ShapeDtypeStruct((NP, H), jnp.float32),
)


def _out_body(p_ref, y2_ref, dinv_ref, b2_ref, wfc_ref, bfc_ref, o_ref):
    dv = dinv_ref[...]
    h = jnp.maximum(dv * (p_ref[0] + p_ref[1] + y2_ref[...]) + b2_ref[...], 0.0)
    logits = _dot(h, wfc_ref[...]) + bfc_ref[...]
    m = jnp.max(logits, axis=1, keepdims=True)
    ex = jnp.exp(logits - m)
    lse = jnp.log(jnp.sum(ex, axis=1, keepdims=True)) + m
    o_ref[...] = logits - lse


_out_call = pl.pallas_call(
    _out_body,
    grid=(_GRID,),
    in_specs=[
        pl.BlockSpec((NC, 128, H), lambda j: (0, j, 0)),
        pl.BlockSpec((128, H), lambda j: (j, 0)),
        pl.BlockSpec((128, 1), lambda j: (j, 0)),
        pl.BlockSpec((1, H), lambda j: (0, 0)),
        pl.BlockSpec((H, C), lambda j: (0, 0)),
        pl.BlockSpec((1, C), lambda j: (0, 0)),
    ],
    out_specs=pl.BlockSpec((128, C), lambda j: (j, 0)),
    out_shape=jax.ShapeDtypeStruct((NP, C), jnp.float32),
)


def kernel(x, edge_index, W1, b1, W2, b2, Wfc, bfc):
    x_pad = jnp.pad(x, ((0, NP - N), (0, 0)))
    src = jnp.pad(edge_index[0], (0, EP - E), constant_values=N)
    dst = jnp.pad(edge_index[1], (0, EP - E), constant_values=N)
    src3 = src.reshape(NW, CPT, CH)
    dst3 = dst.reshape(NW, CPT, CH)
    dst2 = dst.reshape(NW, EPT)
    zeros_np = jnp.zeros((NP, H), jnp.float32)

    deg_parts = _deg_kernel(dst2)
    dinv = _dinv_call(deg_parts)
    dinv_col = dinv.reshape(NP, 1)

    y1 = _y1_call(x_pad, dinv_col, W1)
    p1 = _agg_kernel(y1, src3, dst3, zeros_np)
    y2 = _hy2_call(p1, y1, dinv_col, b1.reshape(1, H), W2)
    p2 = _agg_kernel(y2, src3, dst3, zeros_np)
    out = _out_call(p2, y2, dinv_col, b2.reshape(1, H), Wfc, bfc.reshape(1, C))
    return out[:N]


# speedup vs baseline: 1.0431x; 1.0431x over previous
import functools

import jax
import jax.numpy as jnp
from jax import lax
from jax.experimental import pallas as pl
from jax.experimental.pallas import tpu as pltpu
from jax.experimental.pallas import tpu_sc as plsc

N, D, H, C, E = 10000, 128, 128, 64, 320000

NC, NS = 2, 16
NW = NC * NS
NP = 10112
CH = 128
NBUF = 2
HID = 40
Q0, Q1 = 3, 1
CPT0, CPT1 = Q0 * HID, Q1 * HID
EPT0, EPT1 = CPT0 * CH, CPT1 * CH
E0 = NS * EPT0
EPT = (EPT0 + EPT1) // 2
EP = E0 + NS * EPT1
RPT = NP // NS

_mesh = plsc.VectorSubcoreMesh(
    core_axis_name="c", subcore_axis_name="s", num_cores=NC, num_subcores=NS
)


@functools.partial(
    pl.kernel,
    out_type=jax.ShapeDtypeStruct((NW, NP), jnp.float32),
    mesh=_mesh,
    scratch_types=[
        pltpu.VMEM((EPT,), jnp.int32),
        pltpu.VMEM((NP,), jnp.float32),
    ],
    compiler_params=pltpu.CompilerParams(needs_layout_passes=False),
)
def _deg_kernel(dst_hbm, out_hbm, dst_v, deg_v):
    c = lax.axis_index("c")
    s = lax.axis_index("s")
    wid = s * NC + c
    pltpu.sync_copy(dst_hbm.at[wid], dst_v)

    zeros16 = jnp.zeros((16,), jnp.float32)

    def zero_body(i, carry):
        deg_v[pl.ds(i * 16, 16)] = zeros16
        return carry

    lax.fori_loop(0, NP // 16, zero_body, 0)

    ones16 = jnp.ones((16,), jnp.float32)

    def scat_body(i, carry):
        idx = dst_v[pl.ds(i * 16, 16)]
        plsc.addupdate_scatter(deg_v, [idx], ones16)
        return carry

    lax.fori_loop(0, EPT // 16, scat_body, 0)
    pltpu.sync_copy(deg_v, out_hbm.at[wid])


@functools.partial(
    pl.kernel,
    out_type=jax.ShapeDtypeStruct((NC, NP, H), jnp.float32),
    mesh=_mesh,
    scratch_types=[
        pltpu.VMEM((HID, CH), jnp.int32),
        pltpu.VMEM((HID, CH), jnp.int32),
        pltpu.VMEM((NBUF, CH, H), jnp.float32),
        pltpu.VMEM_SHARED((NP, H), jnp.float32),
        [pltpu.SemaphoreType.DMA] * NBUF,
    ],
)
def _agg_kernel(y_hbm, src0_hbm, dst0_hbm, src1_hbm, dst1_hbm, zeros_hbm,
                out_hbm, src_v, dst_v, rows_v, accum, sems):
    c = lax.axis_index("c")
    s = lax.axis_index("s")
    pltpu.sync_copy(zeros_hbm.at[pl.ds(s * RPT, RPT)],
                    accum.at[pl.ds(s * RPT, RPT)])
    plsc.subcore_barrier()

    def quarter(src_h, dst_h, q):
        pltpu.sync_copy(src_h.at[s, pl.ds(q * HID, HID)], src_v)
        pltpu.sync_copy(dst_h.at[s, pl.ds(q * HID, HID)], dst_v)

        for b in range(NBUF):
            pltpu.async_copy(y_hbm.at[src_v.at[b]], rows_v.at[b], sems[b])

        def body(t, carry):
            for b in range(NBUF):
                j = NBUF * t + b
                pltpu.make_async_copy(y_hbm.at[src_v.at[j]], rows_v.at[b],
                                      sems[b]).wait()
                pltpu.sync_copy(rows_v.at[b], accum.at[dst_v.at[j]], add=True)

                @pl.when(j + NBUF < HID)
                def _():
                    pltpu.async_copy(y_hbm.at[src_v.at[j + NBUF]],
                                     rows_v.at[b], sems[b])

            return carry

        lax.fori_loop(0, HID // NBUF, body, 0)

    @pl.when(c == 0)
    def _():
        for q in range(Q0):
            quarter(src0_hbm, dst0_hbm, q)

    @pl.when(c == 1)
    def _():
        for q in range(Q1):
            quarter(src1_hbm, dst1_hbm, q)

    plsc.subcore_barrier()
    pltpu.sync_copy(accum.at[pl.ds(s * RPT, RPT)],
                    out_hbm.at[c, pl.ds(s * RPT, RPT)])


def _dinv_body(parts_ref, o_ref):
    deg = 1.0 + jnp.sum(parts_ref[...], axis=0)
    o_ref[...] = lax.rsqrt(deg)


_dinv_call = pl.pallas_call(
    _dinv_body,
    out_shape=jax.ShapeDtypeStruct((NP,), jnp.float32),
)

_GRID = NP // 128


def _dot(a, b):
    return jnp.dot(a, b, preferred_element_type=jnp.float32,
                   precision=lax.Precision.HIGHEST)


def _y1_body(x_ref, dinv_ref, w_ref, o_ref):
    o_ref[...] = _dot(x_ref[...] * dinv_ref[...], w_ref[...])


_y1_call = pl.pallas_call(
    _y1_body,
    grid=(_GRID,),
    in_specs=[
        pl.BlockSpec((128, D), lambda j: (j, 0)),
        pl.BlockSpec((128, 1), lambda j: (j, 0)),
        pl.BlockSpec((D, H), lambda j: (0, 0)),
    ],
    out_specs=pl.BlockSpec((128, H), lambda j: (j, 0)),
    out_shape=jax.ShapeDtypeStruct((NP, H), jnp.float32),
)


def _hy2_body(p_ref, y1_ref, dinv_ref, b1_ref, w2_ref, o_ref):
    dv = dinv_ref[...]
    h = jnp.maximum(dv * (p_ref[0] + p_ref[1] + y1_ref[...]) + b1_ref[...], 0.0)
    o_ref[...] = _dot(h * dv, w2_ref[...])


_hy2_call = pl.pallas_call(
    _hy2_body,
    grid=(_GRID,),
    in_specs=[
        pl.BlockSpec((NC, 128, H), lambda j: (0, j, 0)),
        pl.BlockSpec((128, H), lambda j: (j, 0)),
        pl.BlockSpec((128, 1), lambda j: (j, 0)),
        pl.BlockSpec((1, H), lambda j: (0, 0)),
        pl.BlockSpec((H, H), lambda j: (0, 0)),
    ],
    out_specs=pl.BlockSpec((128, H), lambda j: (j, 0)),
    out_shape=jax.ShapeDtypeStruct((NP, H), jnp.float32),
)


def _out_body(p_ref, y2_ref, dinv_ref, b2_ref, wfc_ref, bfc_ref, o_ref):
    dv = dinv_ref[...]
    h = jnp.maximum(dv * (p_ref[0] + p_ref[1] + y2_ref[...]) + b2_ref[...], 0.0)
    logits = _dot(h, wfc_ref[...]) + bfc_ref[...]
    m = jnp.max(logits, axis=1, keepdims=True)
    ex = jnp.exp(logits - m)
    lse = jnp.log(jnp.sum(ex, axis=1, keepdims=True)) + m
    o_ref[...] = logits - lse


_out_call = pl.pallas_call(
    _out_body,
    grid=(_GRID,),
    in_specs=[
        pl.BlockSpec((NC, 128, H), lambda j: (0, j, 0)),
        pl.BlockSpec((128, H), lambda j: (j, 0)),
        pl.BlockSpec((128, 1), lambda j: (j, 0)),
        pl.BlockSpec((1, H), lambda j: (0, 0)),
        pl.BlockSpec((H, C), lambda j: (0, 0)),
        pl.BlockSpec((1, C), lambda j: (0, 0)),
    ],
    out_specs=pl.BlockSpec((128, C), lambda j: (j, 0)),
    out_shape=jax.ShapeDtypeStruct((NP, C), jnp.float32),
)


def kernel(x, edge_index, W1, b1, W2, b2, Wfc, bfc):
    x_pad = jnp.pad(x, ((0, NP - N), (0, 0)))
    src = jnp.pad(edge_index[0], (0, EP - E), constant_values=N)
    dst = jnp.pad(edge_index[1], (0, EP - E), constant_values=N)
    src3a = src[:E0].reshape(NS, CPT0, CH)
    dst3a = dst[:E0].reshape(NS, CPT0, CH)
    src3b = src[E0:].reshape(NS, CPT1, CH)
    dst3b = dst[E0:].reshape(NS, CPT1, CH)
    dst2 = dst.reshape(NW, EPT)
    zeros_np = jnp.zeros((NP, H), jnp.float32)

    deg_parts = _deg_kernel(dst2)
    dinv = _dinv_call(deg_parts)
    dinv_col = dinv.reshape(NP, 1)

    y1 = _y1_call(x_pad, dinv_col, W1)
    p1 = _agg_kernel(y1, src3a, dst3a, src3b, dst3b, zeros_np)
    y2 = _hy2_call(p1, y1, dinv_col, b1.reshape(1, H), W2)
    p2 = _agg_kernel(y2, src3a, dst3a, src3b, dst3b, zeros_np)
    out = _out_call(p2, y2, dinv_col, b2.reshape(1, H), Wfc, bfc.reshape(1, C))
    return out[:N]


# speedup vs baseline: 1.1671x; 1.1188x over previous
import functools

import jax
import jax.numpy as jnp
from jax import lax
from jax.experimental import pallas as pl
from jax.experimental.pallas import tpu as pltpu
from jax.experimental.pallas import tpu_sc as plsc

N, D, H, C, E = 10000, 128, 128, 64, 320000

NC, NS = 2, 16
NW = NC * NS
NP = 10112
HH = H // 2
CH = 128
NBUF = 2
CPT = 160
EPT = CPT * CH
EP = NS * EPT
HID = 40
RPT = NP // NS

_mesh = plsc.VectorSubcoreMesh(
    core_axis_name="c", subcore_axis_name="s", num_cores=NC, num_subcores=NS
)


@functools.partial(
    pl.kernel,
    out_type=jax.ShapeDtypeStruct((NW, NP), jnp.float32),
    mesh=_mesh,
    scratch_types=[
        pltpu.VMEM((EP // NW,), jnp.int32),
        pltpu.VMEM((NP,), jnp.float32),
    ],
    compiler_params=pltpu.CompilerParams(needs_layout_passes=False),
)
def _deg_kernel(dst_hbm, out_hbm, dst_v, deg_v):
    c = lax.axis_index("c")
    s = lax.axis_index("s")
    wid = s * NC + c
    pltpu.sync_copy(dst_hbm.at[wid], dst_v)

    zeros16 = jnp.zeros((16,), jnp.float32)

    def zero_body(i, carry):
        deg_v[pl.ds(i * 16, 16)] = zeros16
        return carry

    lax.fori_loop(0, NP // 16, zero_body, 0)

    ones16 = jnp.ones((16,), jnp.float32)

    def scat_body(i, carry):
        idx = dst_v[pl.ds(i * 16, 16)]
        plsc.addupdate_scatter(deg_v, [idx], ones16)
        return carry

    lax.fori_loop(0, EP // NW // 16, scat_body, 0)
    pltpu.sync_copy(deg_v, out_hbm.at[wid])


@functools.partial(
    pl.kernel,
    out_type=jax.ShapeDtypeStruct((NC, NP, H), jnp.float32),
    mesh=_mesh,
    scratch_types=[
        pltpu.VMEM((HID, CH), jnp.int32),
        pltpu.VMEM((HID, CH), jnp.int32),
        pltpu.VMEM((NBUF, CH, H), jnp.float32),
        pltpu.VMEM_SHARED((NP, H), jnp.float32),
        [pltpu.SemaphoreType.DMA] * NBUF,
    ],
)
def _agg_kernel(y_hbm, src_hbm, dst_hbm, zeros_hbm, out_hbm,
                src_v, dst_v, rows_v, accum, sems):
    c = lax.axis_index("c")
    s = lax.axis_index("s")
    wid = s * NC + c
    rows = pl.ds(s * RPT, RPT)
    pltpu.sync_copy(zeros_hbm.at[rows], accum.at[rows])
    plsc.subcore_barrier()

    for part in range(80 // HID):
        pltpu.sync_copy(src_hbm.at[wid, pl.ds(part * HID, HID)], src_v)
        pltpu.sync_copy(dst_hbm.at[wid, pl.ds(part * HID, HID)], dst_v)

        for b in range(NBUF):
            pltpu.async_copy(y_hbm.at[src_v.at[b]], rows_v.at[b], sems[b])

        def body(t, carry):
            for b in range(NBUF):
                j = NBUF * t + b
                pltpu.make_async_copy(y_hbm.at[src_v.at[j]], rows_v.at[b],
                                      sems[b]).wait()
                pltpu.sync_copy(rows_v.at[b], accum.at[dst_v.at[j]], add=True)

                @pl.when(j + NBUF < HID)
                def _():
                    pltpu.async_copy(y_hbm.at[src_v.at[j + NBUF]],
                                     rows_v.at[b], sems[b])

            return carry

        lax.fori_loop(0, HID // NBUF, body, 0)

    plsc.subcore_barrier()
    pltpu.sync_copy(accum.at[rows], out_hbm.at[c, rows])


def _dinv_body(parts_ref, o_ref):
    deg = 1.0 + jnp.sum(parts_ref[...], axis=0)
    o_ref[...] = lax.rsqrt(deg)


_dinv_call = pl.pallas_call(
    _dinv_body,
    out_shape=jax.ShapeDtypeStruct((NP,), jnp.float32),
)

_BR = 1264
_GRID = NP // _BR


def _dot(a, b):
    return jnp.dot(a, b, preferred_element_type=jnp.float32,
                   precision=lax.Precision.HIGHEST)


def _y1_body(x_ref, dinv_ref, w_ref, o_ref):
    o_ref[...] = _dot(x_ref[...] * dinv_ref[...], w_ref[...])


_y1_call = pl.pallas_call(
    _y1_body,
    grid=(_GRID,),
    in_specs=[
        pl.BlockSpec((_BR, D), lambda j: (j, 0)),
        pl.BlockSpec((_BR, 1), lambda j: (j, 0)),
        pl.BlockSpec((D, H), lambda j: (0, 0)),
    ],
    out_specs=pl.BlockSpec((_BR, H), lambda j: (j, 0)),
    out_shape=jax.ShapeDtypeStruct((NP, H), jnp.float32),
)


def _hy2_body(p_ref, y1_ref, dinv_ref, b1_ref, w2_ref, o_ref):
    dv = dinv_ref[...]
    h = jnp.maximum(dv * (p_ref[0] + p_ref[1] + y1_ref[...]) + b1_ref[...],
                    0.0)
    o_ref[...] = _dot(h * dv, w2_ref[...])


_hy2_call = pl.pallas_call(
    _hy2_body,
    grid=(_GRID,),
    in_specs=[
        pl.BlockSpec((NC, _BR, H), lambda j: (0, j, 0)),
        pl.BlockSpec((_BR, H), lambda j: (j, 0)),
        pl.BlockSpec((_BR, 1), lambda j: (j, 0)),
        pl.BlockSpec((1, H), lambda j: (0, 0)),
        pl.BlockSpec((H, H), lambda j: (0, 0)),
    ],
    out_specs=pl.BlockSpec((_BR, H), lambda j: (j, 0)),
    out_shape=jax.ShapeDtypeStruct((NP, H), jnp.float32),
)


def _out_body(p_ref, y2_ref, dinv_ref, b2_ref, wfc_ref, bfc_ref, o_ref):
    dv = dinv_ref[...]
    h = jnp.maximum(dv * (p_ref[0] + p_ref[1] + y2_ref[...]) + b2_ref[...],
                    0.0)
    logits = _dot(h, wfc_ref[...]) + bfc_ref[...]
    m = jnp.max(logits, axis=1, keepdims=True)
    ex = jnp.exp(logits - m)
    lse = jnp.log(jnp.sum(ex, axis=1, keepdims=True)) + m
    o_ref[...] = logits - lse


_out_call = pl.pallas_call(
    _out_body,
    grid=(_GRID,),
    in_specs=[
        pl.BlockSpec((NC, _BR, H), lambda j: (0, j, 0)),
        pl.BlockSpec((_BR, H), lambda j: (j, 0)),
        pl.BlockSpec((_BR, 1), lambda j: (j, 0)),
        pl.BlockSpec((1, H), lambda j: (0, 0)),
        pl.BlockSpec((H, C), lambda j: (0, 0)),
        pl.BlockSpec((1, C), lambda j: (0, 0)),
    ],
    out_specs=pl.BlockSpec((_BR, C), lambda j: (j, 0)),
    out_shape=jax.ShapeDtypeStruct((NP, C), jnp.float32),
)


def kernel(x, edge_index, W1, b1, W2, b2, Wfc, bfc):
    x_pad = jnp.pad(x, ((0, NP - N), (0, 0)))
    src = jnp.pad(edge_index[0], (0, EP - E), constant_values=N)
    dst = jnp.pad(edge_index[1], (0, EP - E), constant_values=N)
    src3 = src.reshape(NW, 80, CH)
    dst3 = dst.reshape(NW, 80, CH)
    dst2 = dst.reshape(NW, EP // NW)
    zeros_np = jnp.zeros((NP, H), jnp.float32)

    deg_parts = _deg_kernel(dst2)
    dinv = _dinv_call(deg_parts)
    dinv_col = dinv.reshape(NP, 1)

    y1 = _y1_call(x_pad, dinv_col, W1)
    p1 = _agg_kernel(y1, src3, dst3, zeros_np)
    y2 = _hy2_call(p1, y1, dinv_col, b1.reshape(1, H), W2)
    p2 = _agg_kernel(y2, src3, dst3, zeros_np)
    out = _out_call(p2, y2, dinv_col, b2.reshape(1, H), Wfc, bfc.reshape(1, C))
    return out[:N]
